# baseline (speedup 1.0000x reference)
import dataclasses
import functools

import jax
import jax.numpy as jnp
from jax import lax
from jax.experimental import pallas as pl
from jax.experimental.pallas import tpu as pltpu
from jax.experimental.pallas import tpu_sc as plsc

_NUM_EXPERTS = 64
_TOP_K = 8
_ALPHA = 0.01

_SC_CORES = 2
_SC_SUBCORES = 16
_LANES = 16
_NW = _SC_CORES * _SC_SUBCORES


def _sc_hist_body(idx_hbm, out_hbm, idx_v, hist_v, cnt_v, sem):
    n = idx_v.shape[0]
    wid = lax.axis_index("s") * _SC_CORES + lax.axis_index("c")
    cp = pltpu.make_async_copy(idx_hbm.at[wid], idx_v, sem)
    cp.start()

    zeros = jnp.zeros((_LANES,), jnp.int32)

    @pl.loop(0, _LANES * _NUM_EXPERTS, step=_LANES)
    def _(j):
        hist_v[pl.ds(j, _LANES)] = zeros

    cp.wait()

    lane_base = lax.iota(jnp.int32, _LANES) * _NUM_EXPERTS
    ones = jnp.ones((_LANES,), jnp.int32)

    @pl.loop(0, n, step=_LANES)
    def _(i):
        idx = idx_v[pl.ds(i, _LANES)]
        plsc.addupdate_scatter(hist_v, [lane_base + idx], ones)

    for j in range(0, _NUM_EXPERTS, _LANES):
        acc = hist_v[pl.ds(j, _LANES)]
        for r in range(1, _LANES):
            acc = acc + hist_v[pl.ds(r * _NUM_EXPERTS + j, _LANES)]
        cnt_v[pl.ds(j, _LANES)] = acc

    pltpu.sync_copy(cnt_v, out_hbm.at[wid])


def _sc_bincount(idx_rows):
    n = idx_rows.shape[1]
    mesh = plsc.VectorSubcoreMesh(
        core_axis_name="c", subcore_axis_name="s",
        num_cores=_SC_CORES, num_subcores=_SC_SUBCORES,
    )
    cp = pltpu.CompilerParams()
    if "needs_layout_passes" in pltpu.CompilerParams.__dataclass_fields__:
        cp = dataclasses.replace(cp, needs_layout_passes=False)
    kern = pl.kernel(
        _sc_hist_body,
        out_type=jax.ShapeDtypeStruct((_NW, _NUM_EXPERTS), jnp.int32),
        mesh=mesh,
        compiler_params=cp,
        scratch_types=[
            pltpu.VMEM((n,), jnp.int32),
            pltpu.VMEM((_LANES * _NUM_EXPERTS,), jnp.int32),
            pltpu.VMEM((_NUM_EXPERTS,), jnp.int32),
            pltpu.SemaphoreType.DMA,
        ],
    )
    return kern(idx_rows)


def _tc_loss_body(scale, cnt_ref, x_ref, o_ref, acc_ref):
    nb = pl.num_programs(0)
    nj = pl.num_programs(1)
    b = pl.program_id(0)
    j = pl.program_id(1)
    step = b * nj + j

    x = x_ref[0]
    m = jnp.max(x, axis=0, keepdims=True)
    e = jnp.exp(x - m)
    s = jnp.sum(e, axis=0, keepdims=True)
    p = e * (1.0 / s)
    part = p[:, 0:128]
    for c in range(1, p.shape[1] // 128):
        part = part + p[:, c * 128:(c + 1) * 128]

    @pl.when(step == 0)
    def _():
        acc_ref[...] = part

    @pl.when(step != 0)
    def _():
        acc_ref[...] = acc_ref[...] + part

    @pl.when(step == nb * nj - 1)
    def _():
        cvec = jnp.sum(cnt_ref[...].astype(jnp.float32), axis=0,
                       keepdims=True)
        r = jax.lax.dot_general(
            cvec, acc_ref[...],
            dimension_numbers=(((1,), (0,)), ((), ())),
            preferred_element_type=jnp.float32,
        )
        o_ref[...] = jnp.sum(r, axis=1, keepdims=True) * scale


def _tc_loss(xt, partial_counts, seq_blk, scale):
    batch, ne, seq = xt.shape
    grid = (batch, seq // seq_blk)
    out = pl.pallas_call(
        functools.partial(_tc_loss_body, scale),
        grid=grid,
        in_specs=[
            pl.BlockSpec((_NW, ne), lambda b, j: (0, 0)),
            pl.BlockSpec((1, ne, seq_blk), lambda b, j: (b, 0, j)),
        ],
        out_specs=pl.BlockSpec((1, 1), lambda b, j: (0, 0)),
        out_shape=jax.ShapeDtypeStruct((1, 1), jnp.float32),
        scratch_shapes=[pltpu.VMEM((ne, 128), jnp.float32)],
        compiler_params=pltpu.CompilerParams(
            dimension_semantics=("arbitrary", "arbitrary"),
        ),
    )(partial_counts, xt)
    return out


@jax.jit
def kernel(router_logits, expert_indices):
    batch, seq, _ = router_logits.shape
    num_tokens = batch * seq
    xt = jnp.transpose(router_logits, (0, 2, 1))
    idx_rows = jnp.transpose(expert_indices, (0, 2, 1)).reshape(_NW, -1)

    partial_counts = _sc_bincount(idx_rows)
    scale = _ALPHA * _NUM_EXPERTS * _NUM_EXPERTS / (
        _TOP_K * float(num_tokens) * float(num_tokens))
    out = _tc_loss(xt, partial_counts, seq_blk=2048, scale=scale)
    return out.reshape(())


# speedup vs baseline: 4.3733x; 4.3733x over previous
import functools

import jax
import jax.numpy as jnp
from jax.experimental import pallas as pl
from jax.experimental.pallas import tpu as pltpu

_NUM_EXPERTS = 64
_TOP_K = 8
_ALPHA = 0.01

_NIBBLE_GROUP = 14


def _loss_body(scale, x_ref, idx_ref, o_ref, acc_ref, cnt_ref, l2_ref):
    nb = pl.num_programs(0)
    b = pl.program_id(0)

    x = x_ref[0]
    m = jnp.max(x, axis=0, keepdims=True)
    e = jnp.exp(x - m)
    s = jnp.sum(e, axis=0, keepdims=True)
    p = e * (1.0 / s)
    part = p[:, 0:128]
    for c in range(1, p.shape[1] // 128):
        part = part + p[:, c * 128:(c + 1) * 128]

    @pl.when(b == 0)
    def _():
        acc_ref[...] = part
        cnt_ref[...] = jnp.zeros_like(cnt_ref)

    @pl.when(b != 0)
    def _():
        acc_ref[...] = acc_ref[...] + part

    l2_ref[...] = jnp.zeros_like(l2_ref)

    idx = idx_ref[0]
    nvec = idx.shape[1] // 128
    hvals = jnp.arange(8, dtype=jnp.int32)
    for g0 in range(0, nvec, _NIBBLE_GROUP):
        g1 = min(g0 + _NIBBLE_GROUP, nvec)
        accs = [jnp.zeros((8, 128), jnp.int32) for _ in range(8)]
        for i in range(g0, g1):
            blk = idx[:, i * 128:(i + 1) * 128]
            hi = blk >> 3
            lo = blk & 7
            pw = jnp.left_shift(jnp.int32(1), lo << 2)
            for h in range(8):
                accs[h] = accs[h] + jnp.where(hi == hvals[h], pw, 0)
        for h in range(8):
            l2_ref[h, 0] = l2_ref[h, 0] + (accs[h] & 0x0F0F0F0F)
            l2_ref[h, 1] = l2_ref[h, 1] + ((accs[h] >> 4) & 0x0F0F0F0F)

    for h in range(8):
        for par in range(2):
            w = l2_ref[h, par]
            for b4 in range(4):
                lo_val = 2 * b4 + par
                cnt8 = (w >> (8 * b4)) & 0xFF
                row = 8 * h + lo_val
                cnt_ref[row:row + 1, :] = (
                    cnt_ref[row:row + 1, :]
                    + jnp.sum(cnt8, axis=0, keepdims=True))

    @pl.when(b == nb - 1)
    def _():
        psum = jnp.sum(acc_ref[...], axis=1, keepdims=True)
        csum = jnp.sum(cnt_ref[...].astype(jnp.float32), axis=1,
                       keepdims=True)
        o_ref[...] = jnp.sum(psum * csum, axis=0, keepdims=True) * scale


@jax.jit
def kernel(router_logits, expert_indices):
    batch, seq, ne = router_logits.shape
    num_tokens = batch * seq
    xt = jnp.transpose(router_logits, (0, 2, 1))
    it = jnp.transpose(expert_indices, (0, 2, 1))

    scale = _ALPHA * _NUM_EXPERTS * _NUM_EXPERTS / (
        _TOP_K * float(num_tokens) * float(num_tokens))

    out = pl.pallas_call(
        functools.partial(_loss_body, scale),
        grid=(batch,),
        in_specs=[
            pl.BlockSpec((1, ne, seq), lambda b: (b, 0, 0)),
            pl.BlockSpec((1, _TOP_K, seq), lambda b: (b, 0, 0)),
        ],
        out_specs=pl.BlockSpec((1, 1), lambda b: (0, 0)),
        out_shape=jax.ShapeDtypeStruct((1, 1), jnp.float32),
        scratch_shapes=[
            pltpu.VMEM((_NUM_EXPERTS, 128), jnp.float32),
            pltpu.VMEM((_NUM_EXPERTS, 128), jnp.int32),
            pltpu.VMEM((8, 2, 8, 128), jnp.int32),
        ],
        compiler_params=pltpu.CompilerParams(
            dimension_semantics=("arbitrary",),
        ),
    )(xt, it)
    return out.reshape(())


# speedup vs baseline: 4.9255x; 1.1263x over previous
import functools

import jax
import jax.numpy as jnp
from jax.experimental import pallas as pl
from jax.experimental.pallas import tpu as pltpu

_NUM_EXPERTS = 64
_TOP_K = 8
_ALPHA = 0.01

_NIBBLE_GROUP = 14


def _loss_body(scale, x_ref, idx_ref, o_ref, acc_ref, cnt_ref, l2_ref):
    nb = pl.num_programs(0)
    b = pl.program_id(0)

    x = x_ref[0]
    m = jnp.max(x, axis=0, keepdims=True)
    e = jnp.exp(x - m)
    ones = jnp.ones((1, x.shape[0]), jnp.float32)
    s = jax.lax.dot_general(
        ones, e, (((1,), (0,)), ((), ())),
        preferred_element_type=jnp.float32)
    r = 1.0 / s
    part = jax.lax.dot_general(
        e, r, (((1,), (1,)), ((), ())),
        preferred_element_type=jnp.float32)

    @pl.when(b == 0)
    def _():
        acc_ref[...] = jnp.zeros_like(acc_ref)
        cnt_ref[...] = jnp.zeros_like(cnt_ref)

    acc_ref[:, 0:1] = acc_ref[:, 0:1] + part

    l2_ref[...] = jnp.zeros_like(l2_ref)

    idx = idx_ref[0]
    nvec = idx.shape[1] // 128
    hvals = jnp.arange(8, dtype=jnp.int32)
    for g0 in range(0, nvec, _NIBBLE_GROUP):
        g1 = min(g0 + _NIBBLE_GROUP, nvec)
        accs = [jnp.zeros((8, 128), jnp.int32) for _ in range(8)]
        for i in range(g0, g1):
            blk = idx[:, i * 128:(i + 1) * 128]
            hi = blk >> 3
            lo = blk & 7
            pw = jnp.left_shift(jnp.int32(1), lo << 2)
            for h in range(8):
                accs[h] = accs[h] + jnp.where(hi == hvals[h], pw, 0)
        for h in range(8):
            l2_ref[h, 0] = l2_ref[h, 0] + (accs[h] & 0x0F0F0F0F)
            l2_ref[h, 1] = l2_ref[h, 1] + ((accs[h] >> 4) & 0x0F0F0F0F)

    for h in range(8):
        for par in range(2):
            w = l2_ref[h, par]
            for b4 in range(4):
                lo_val = 2 * b4 + par
                cnt8 = (w >> (8 * b4)) & 0xFF
                row = 8 * h + lo_val
                cnt_ref[row:row + 1, :] = (
                    cnt_ref[row:row + 1, :]
                    + jnp.sum(cnt8, axis=0, keepdims=True))

    @pl.when(b == nb - 1)
    def _():
        psum = jnp.sum(acc_ref[...], axis=1, keepdims=True)
        csum = jnp.sum(cnt_ref[...].astype(jnp.float32), axis=1,
                       keepdims=True)
        o_ref[...] = jnp.sum(psum * csum, axis=0, keepdims=True) * scale


@jax.jit
def kernel(router_logits, expert_indices):
    batch, seq, ne = router_logits.shape
    num_tokens = batch * seq
    xt = jnp.transpose(router_logits, (0, 2, 1))
    it = jnp.transpose(expert_indices, (0, 2, 1))

    scale = _ALPHA * _NUM_EXPERTS * _NUM_EXPERTS / (
        _TOP_K * float(num_tokens) * float(num_tokens))

    out = pl.pallas_call(
        functools.partial(_loss_body, scale),
        grid=(batch,),
        in_specs=[
            pl.BlockSpec((1, ne, seq), lambda b: (b, 0, 0)),
            pl.BlockSpec((1, _TOP_K, seq), lambda b: (b, 0, 0)),
        ],
        out_specs=pl.BlockSpec((1, 1), lambda b: (0, 0)),
        out_shape=jax.ShapeDtypeStruct((1, 1), jnp.float32),
        scratch_shapes=[
            pltpu.VMEM((_NUM_EXPERTS, 128), jnp.float32),
            pltpu.VMEM((_NUM_EXPERTS, 128), jnp.int32),
            pltpu.VMEM((8, 2, 8, 128), jnp.int32),
        ],
        compiler_params=pltpu.CompilerParams(
            dimension_semantics=("arbitrary",),
        ),
    )(xt, it)
    return out.reshape(())
